# Initial kernel scaffold; baseline (speedup 1.0000x reference)
import functools

import jax
import jax.numpy as jnp
from jax import lax
from jax.experimental import pallas as pl
from jax.experimental.pallas import tpu as pltpu
from jax.experimental.pallas import tpu_sc as plsc

N = 10000
E = 320000
NF = 128
NH = 128
NEG_SLOPE = 0.05

NC = 2
NS = 16
NW = NC * NS
EPW = E // NW
NPAD = 10240
CHK = NPAD // NS
K = 80
SB = K // 16
NBLK = EPW // K

_MESH = plsc.VectorSubcoreMesh(core_axis_name="c", subcore_axis_name="s",
                               num_cores=NC, num_subcores=NS)
_Z16 = functools.partial(jnp.zeros, (16,), jnp.float32)


def _tc_pre(x_ref, w_ref, awl_ref, awr_ref, ab_ref, h_ref, hl_ref, hr_ref):
    h = jnp.dot(x_ref[...], w_ref[...], preferred_element_type=jnp.float32)
    h_ref[...] = h
    hl_ref[...] = jnp.sum(h * awl_ref[...], axis=1, keepdims=True) + ab_ref[0]
    hr_ref[...] = jnp.sum(h * awr_ref[...], axis=1, keepdims=True)


def _sc_att(src_hbm, dst_hbm, hl_hbm, hr_hbm, e_hbm, hsum_hbm,
            src_v, dst_v, hl_v, hr_v, e_v, hsum_v, tmp_v, acc_v, hstage_sh):
    cid = lax.axis_index("c")
    sid = lax.axis_index("s")
    wid = cid * NS + sid
    base = wid * EPW
    pltpu.sync_copy(src_hbm.at[pl.ds(base, EPW)], src_v)
    pltpu.sync_copy(dst_hbm.at[pl.ds(base, EPW)], dst_v)
    pltpu.sync_copy(hl_hbm, hl_v)
    pltpu.sync_copy(hr_hbm, hr_v)

    def zero_hsum(k, carry):
        hsum_v[pl.ds(k * 16, 16)] = _Z16()
        return carry
    lax.fori_loop(0, NPAD // 16, zero_hsum, 0)

    def step(i, carry):
        sl = pl.ds(i * 16, 16)
        s16 = src_v[sl]
        d16 = dst_v[sl]
        t = plsc.load_gather(hl_v, [s16]) + plsc.load_gather(hr_v, [d16])
        t = jnp.where(t > 0.0, t, t * NEG_SLOPE)
        e16 = jnp.exp(t)
        e_v[sl] = e16
        plsc.addupdate_scatter(hsum_v, [s16], e16)
        return carry
    lax.fori_loop(0, EPW // 16, step, 0)

    pltpu.sync_copy(e_v, e_hbm.at[pl.ds(base, EPW)])
    pltpu.sync_copy(hsum_v, hstage_sh.at[sid])
    plsc.subcore_barrier()

    def zero_acc(k, carry):
        acc_v[pl.ds(k * 16, 16)] = _Z16()
        return carry
    lax.fori_loop(0, CHK // 16, zero_acc, 0)
    for r in range(NS):
        pltpu.sync_copy(hstage_sh.at[r, pl.ds(sid * CHK, CHK)], tmp_v)

        def addc(k, carry):
            sl = pl.ds(k * 16, 16)
            acc_v[sl] = acc_v[sl] + tmp_v[sl]
            return carry
        lax.fori_loop(0, CHK // 16, addc, 0)
    pltpu.sync_copy(acc_v, hsum_hbm.at[cid, pl.ds(sid * CHK, CHK)])


def _sc_agg(h_hbm, src_hbm, dstr_hbm, e_hbm, adj_hbm, hsum_hbm,
            alpha_hbm, outp_hbm,
            src_v, dst2_v, e_v, adj_v, hs_v, hp_v, alpha_v, valb_v,
            rowbuf_v, out_sh):
    cid = lax.axis_index("c")
    sid = lax.axis_index("s")
    wid = cid * NS + sid
    base = wid * EPW
    pltpu.sync_copy(src_hbm.at[pl.ds(base, EPW)], src_v)
    pltpu.sync_copy(dstr_hbm.at[wid], dst2_v)
    pltpu.sync_copy(e_hbm.at[pl.ds(base, EPW)], e_v)
    pltpu.sync_copy(adj_hbm.at[pl.ds(base, EPW)], adj_v)
    pltpu.sync_copy(hsum_hbm.at[0], hs_v)
    pltpu.sync_copy(hsum_hbm.at[1], hp_v)

    def addp(k, carry):
        sl = pl.ds(k * 16, 16)
        hs_v[sl] = hs_v[sl] + hp_v[sl]
        return carry
    lax.fori_loop(0, NPAD // 16, addp, 0)

    def zrow(r, carry):
        for c in range(NH // 16):
            rowbuf_v[r, pl.ds(c * 16, 16)] = _Z16()
        return carry
    lax.fori_loop(0, K, zrow, 0)
    for t in range(CHK // K):
        pltpu.sync_copy(rowbuf_v, out_sh.at[pl.ds(sid * CHK + t * K, K)])
    plsc.subcore_barrier()

    def blk(j, carry):
        pltpu.sync_copy(h_hbm.at[dst2_v.at[j]], rowbuf_v)
        for sb in range(SB):
            sl = pl.ds(j * K + sb * 16, 16)
            s16 = src_v[sl]
            al16 = e_v[sl] / plsc.load_gather(hs_v, [s16])
            alpha_v[sl] = al16
            valb_v[...] = al16 * adj_v[sl]
            for r in range(16):
                sp = plsc.load_gather(valb_v, [jnp.full((16,), r, jnp.int32)])
                rr = sb * 16 + r
                for c in range(NH // 16):
                    csl = pl.ds(c * 16, 16)
                    rowbuf_v[rr, csl] = rowbuf_v[rr, csl] * sp
            pltpu.sync_copy(rowbuf_v.at[pl.ds(sb * 16, 16)], out_sh.at[s16],
                            add=True)
        return carry
    lax.fori_loop(0, NBLK, blk, 0)

    pltpu.sync_copy(alpha_v, alpha_hbm.at[pl.ds(base, EPW)])
    plsc.subcore_barrier()
    pltpu.sync_copy(out_sh.at[pl.ds(sid * CHK, CHK)],
                    outp_hbm.at[cid, pl.ds(sid * CHK, CHK)])


def _tc_fin(p_ref, o_ref):
    o_ref[...] = p_ref[0, :N, :] + p_ref[1, :N, :]


@jax.jit
def kernel(x, edge_index, adj_vals, W_fc, a_W, a_b):
    src = edge_index[0]
    dst = edge_index[1]
    awl = a_W[:NH, 0].reshape(1, NH)
    awr = a_W[NH:, 0].reshape(1, NH)

    h, hl2, hr2 = pl.pallas_call(
        _tc_pre,
        out_shape=[
            jax.ShapeDtypeStruct((N, NH), jnp.float32),
            jax.ShapeDtypeStruct((N, 1), jnp.float32),
            jax.ShapeDtypeStruct((N, 1), jnp.float32),
        ],
        in_specs=[
            pl.BlockSpec(memory_space=pltpu.MemorySpace.VMEM),
            pl.BlockSpec(memory_space=pltpu.MemorySpace.VMEM),
            pl.BlockSpec(memory_space=pltpu.MemorySpace.VMEM),
            pl.BlockSpec(memory_space=pltpu.MemorySpace.VMEM),
            pl.BlockSpec(memory_space=pltpu.MemorySpace.SMEM),
        ],
    )(x, W_fc, awl, awr, a_b)
    hl = hl2.reshape(N)
    hr = hr2.reshape(N)

    e, hsum = pl.kernel(
        _sc_att,
        out_type=[
            jax.ShapeDtypeStruct((E,), jnp.float32),
            jax.ShapeDtypeStruct((NC, NPAD), jnp.float32),
        ],
        mesh=_MESH,
        scratch_types=[
            pltpu.VMEM((EPW,), jnp.int32),
            pltpu.VMEM((EPW,), jnp.int32),
            pltpu.VMEM((N,), jnp.float32),
            pltpu.VMEM((N,), jnp.float32),
            pltpu.VMEM((EPW,), jnp.float32),
            pltpu.VMEM((NPAD,), jnp.float32),
            pltpu.VMEM((CHK,), jnp.float32),
            pltpu.VMEM((CHK,), jnp.float32),
            pltpu.MemorySpace.VMEM_SHARED((NS, NPAD), jnp.float32),
        ],
    )(src, dst, hl, hr)

    dst_r = dst.reshape(NW, NBLK, K)
    alpha, outp = pl.kernel(
        _sc_agg,
        out_type=[
            jax.ShapeDtypeStruct((E,), jnp.float32),
            jax.ShapeDtypeStruct((NC, NPAD, NH), jnp.float32),
        ],
        mesh=_MESH,
        scratch_types=[
            pltpu.VMEM((EPW,), jnp.int32),
            pltpu.VMEM((NBLK, K), jnp.int32),
            pltpu.VMEM((EPW,), jnp.float32),
            pltpu.VMEM((EPW,), jnp.float32),
            pltpu.VMEM((NPAD,), jnp.float32),
            pltpu.VMEM((NPAD,), jnp.float32),
            pltpu.VMEM((EPW,), jnp.float32),
            pltpu.VMEM((16,), jnp.float32),
            pltpu.VMEM((K, NH), jnp.float32),
            pltpu.MemorySpace.VMEM_SHARED((NPAD, NH), jnp.float32),
        ],
    )(h, src, dst_r, e, adj_vals, hsum)

    out = pl.pallas_call(
        _tc_fin,
        out_shape=jax.ShapeDtypeStruct((N, NH), jnp.float32),
    )(outp)
    return (out, alpha)


# speedup vs baseline: 13.0890x; 13.0890x over previous
import functools

import jax
import jax.numpy as jnp
from jax import lax
from jax.experimental import pallas as pl
from jax.experimental.pallas import tpu as pltpu
from jax.experimental.pallas import tpu_sc as plsc

N = 10000
E = 320000
NF = 128
NH = 128
NEG_SLOPE = 0.05

NC = 2
NS = 16
NW = NC * NS
EPW = E // NW
NPAD = 10240
CHK = NPAD // NS
K = 80
SB = K // 16
NR = 5
ROUND = EPW // NR
NBLKR = ROUND // K

_MESH = plsc.VectorSubcoreMesh(core_axis_name="c", subcore_axis_name="s",
                               num_cores=NC, num_subcores=NS)
_SC_PARAMS = pltpu.CompilerParams(needs_layout_passes=False)
_Z16 = functools.partial(jnp.zeros, (16,), jnp.float32)


def _tc_pre(x_ref, w_ref, awl_ref, awr_ref, ab_ref, h_ref, hl_ref, hr_ref):
    h = jnp.dot(x_ref[...], w_ref[...], preferred_element_type=jnp.float32)
    h_ref[...] = h
    hl_ref[...] = jnp.sum(h * awl_ref[...], axis=1, keepdims=True) + ab_ref[0]
    hr_ref[...] = jnp.sum(h * awr_ref[...], axis=1, keepdims=True)


def _sc_att(src_hbm, dst_hbm, hl_hbm, hr_hbm, e_hbm, hsum_hbm,
            src_v, dst_v, hl_v, hr_v, e_v, hsum_v, tmp_v, acc_v, hstage_sh):
    cid = lax.axis_index("c")
    sid = lax.axis_index("s")
    wid = cid * NS + sid
    base = wid * EPW
    pltpu.sync_copy(src_hbm.at[pl.ds(base, EPW)], src_v)
    pltpu.sync_copy(dst_hbm.at[pl.ds(base, EPW)], dst_v)
    pltpu.sync_copy(hl_hbm, hl_v)
    pltpu.sync_copy(hr_hbm, hr_v)

    def zero_hsum(k, carry):
        hsum_v[pl.ds(k * 16, 16)] = _Z16()
        return carry
    lax.fori_loop(0, NPAD // 16, zero_hsum, 0)

    def step(i, carry):
        sl = pl.ds(i * 16, 16)
        s16 = src_v[sl]
        d16 = dst_v[sl]
        t = plsc.load_gather(hl_v, [s16]) + plsc.load_gather(hr_v, [d16])
        t = jnp.where(t > 0.0, t, t * NEG_SLOPE)
        e16 = jnp.exp(t)
        e_v[sl] = e16
        plsc.addupdate_scatter(hsum_v, [s16], e16)
        return carry
    lax.fori_loop(0, EPW // 16, step, 0)

    pltpu.sync_copy(e_v, e_hbm.at[pl.ds(base, EPW)])
    pltpu.sync_copy(hsum_v, hstage_sh.at[sid])
    plsc.subcore_barrier()

    def zero_acc(k, carry):
        acc_v[pl.ds(k * 16, 16)] = _Z16()
        return carry
    lax.fori_loop(0, CHK // 16, zero_acc, 0)
    for r in range(NS):
        pltpu.sync_copy(hstage_sh.at[r, pl.ds(sid * CHK, CHK)], tmp_v)

        def addc(k, carry):
            sl = pl.ds(k * 16, 16)
            acc_v[sl] = acc_v[sl] + tmp_v[sl]
            return carry
        lax.fori_loop(0, CHK // 16, addc, 0)
    pltpu.sync_copy(acc_v, hsum_hbm.at[cid, pl.ds(sid * CHK, CHK)])


def _sc_agg(h_hbm, src_hbm, dstr_hbm, e_hbm, adj_hbm, hsum_hbm,
            alpha_hbm, outp_hbm,
            src_v, dst2_v, e_v, adj_v, hs_v, tmp_v, alpha_v,
            rowbuf_v, out_sh):
    cid = lax.axis_index("c")
    sid = lax.axis_index("s")
    wid = cid * NS + sid
    base = wid * EPW

    pltpu.sync_copy(hsum_hbm.at[0], hs_v)
    for t in range(NPAD // CHK):
        pltpu.sync_copy(hsum_hbm.at[1, pl.ds(t * CHK, CHK)], tmp_v)

        def addp(k, carry, t=t):
            sl = pl.ds(t * CHK + k * 16, 16)
            sl2 = pl.ds(k * 16, 16)
            hs_v[sl] = hs_v[sl] + tmp_v[sl2]
            return carry
        lax.fori_loop(0, CHK // 16, addp, 0)

    def zrow(r, carry):
        for c in range(NH // 16):
            rowbuf_v[r, pl.ds(c * 16, 16)] = _Z16()
        return carry
    lax.fori_loop(0, K, zrow, 0)
    for t in range(CHK // K):
        pltpu.sync_copy(rowbuf_v, out_sh.at[pl.ds(sid * CHK + t * K, K)])
    plsc.subcore_barrier()

    def rnd(r, carry):
        rbase = base + r * ROUND
        pltpu.sync_copy(src_hbm.at[pl.ds(rbase, ROUND)], src_v)
        pltpu.sync_copy(dstr_hbm.at[wid, r], dst2_v)
        pltpu.sync_copy(e_hbm.at[pl.ds(rbase, ROUND)], e_v)
        pltpu.sync_copy(adj_hbm.at[pl.ds(rbase, ROUND)], adj_v)

        def blk(j, carry):
            pltpu.sync_copy(h_hbm.at[dst2_v.at[j]], rowbuf_v)
            for sb in range(SB):
                sl = pl.ds(j * K + sb * 16, 16)
                s16 = src_v[sl]
                al16 = e_v[sl] / plsc.load_gather(hs_v, [s16])
                alpha_v[sl] = al16
                val16 = al16 * adj_v[sl]
                for rr16 in range(16):
                    sp = jnp.broadcast_to(val16[rr16], (16,))
                    rr = sb * 16 + rr16
                    for c in range(NH // 16):
                        csl = pl.ds(c * 16, 16)
                        rowbuf_v[rr, csl] = rowbuf_v[rr, csl] * sp
                pltpu.sync_copy(rowbuf_v.at[pl.ds(sb * 16, 16)],
                                out_sh.at[s16], add=True)
            return carry
        lax.fori_loop(0, NBLKR, blk, 0)
        pltpu.sync_copy(alpha_v, alpha_hbm.at[pl.ds(rbase, ROUND)])
        return carry
    lax.fori_loop(0, NR, rnd, 0)

    plsc.subcore_barrier()
    pltpu.sync_copy(out_sh.at[pl.ds(sid * CHK, CHK)],
                    outp_hbm.at[cid, pl.ds(sid * CHK, CHK)])


def _tc_fin(p_ref, o_ref):
    o_ref[...] = p_ref[0, :N, :] + p_ref[1, :N, :]


@jax.jit
def kernel(x, edge_index, adj_vals, W_fc, a_W, a_b):
    src = edge_index[0]
    dst = edge_index[1]
    awl = a_W[:NH, 0].reshape(1, NH)
    awr = a_W[NH:, 0].reshape(1, NH)

    h, hl2, hr2 = pl.pallas_call(
        _tc_pre,
        out_shape=[
            jax.ShapeDtypeStruct((N, NH), jnp.float32),
            jax.ShapeDtypeStruct((N, 1), jnp.float32),
            jax.ShapeDtypeStruct((N, 1), jnp.float32),
        ],
        in_specs=[
            pl.BlockSpec(memory_space=pltpu.MemorySpace.VMEM),
            pl.BlockSpec(memory_space=pltpu.MemorySpace.VMEM),
            pl.BlockSpec(memory_space=pltpu.MemorySpace.VMEM),
            pl.BlockSpec(memory_space=pltpu.MemorySpace.VMEM),
            pl.BlockSpec(memory_space=pltpu.MemorySpace.SMEM),
        ],
    )(x, W_fc, awl, awr, a_b)
    hl = hl2.reshape(N)
    hr = hr2.reshape(N)

    e, hsum = pl.kernel(
        _sc_att,
        out_type=[
            jax.ShapeDtypeStruct((E,), jnp.float32),
            jax.ShapeDtypeStruct((NC, NPAD), jnp.float32),
        ],
        mesh=_MESH,
        compiler_params=_SC_PARAMS,
        scratch_types=[
            pltpu.VMEM((EPW,), jnp.int32),
            pltpu.VMEM((EPW,), jnp.int32),
            pltpu.VMEM((N,), jnp.float32),
            pltpu.VMEM((N,), jnp.float32),
            pltpu.VMEM((EPW,), jnp.float32),
            pltpu.VMEM((NPAD,), jnp.float32),
            pltpu.VMEM((CHK,), jnp.float32),
            pltpu.VMEM((CHK,), jnp.float32),
            pltpu.MemorySpace.VMEM_SHARED((NS, NPAD), jnp.float32),
        ],
    )(src, dst, hl, hr)

    dst_r = dst.reshape(NW, NR, NBLKR, K)
    alpha, outp = pl.kernel(
        _sc_agg,
        out_type=[
            jax.ShapeDtypeStruct((E,), jnp.float32),
            jax.ShapeDtypeStruct((NC, NPAD, NH), jnp.float32),
        ],
        mesh=_MESH,
        compiler_params=_SC_PARAMS,
        scratch_types=[
            pltpu.VMEM((ROUND,), jnp.int32),
            pltpu.VMEM((NBLKR, K), jnp.int32),
            pltpu.VMEM((ROUND,), jnp.float32),
            pltpu.VMEM((ROUND,), jnp.float32),
            pltpu.VMEM((NPAD,), jnp.float32),
            pltpu.VMEM((CHK,), jnp.float32),
            pltpu.VMEM((ROUND,), jnp.float32),
            pltpu.VMEM((K, NH), jnp.float32),
            pltpu.MemorySpace.VMEM_SHARED((NPAD, NH), jnp.float32),
        ],
    )(h, src, dst_r, e, adj_vals, hsum)

    out = pl.pallas_call(
        _tc_fin,
        out_shape=jax.ShapeDtypeStruct((N, NH), jnp.float32),
    )(outp)
    return (out, alpha)


# speedup vs baseline: 19.0339x; 1.4542x over previous
import functools

import jax
import jax.numpy as jnp
from jax import lax
from jax.experimental import pallas as pl
from jax.experimental.pallas import tpu as pltpu
from jax.experimental.pallas import tpu_sc as plsc

N = 10000
E = 320000
NF = 128
NH = 128
NEG_SLOPE = 0.05

NC = 2
NS = 16
NW = NC * NS
EPW = E // NW
NPAD = 10240
CHK = NPAD // NS
K = 80
SB = K // 16
NR = 5
ROUND = EPW // NR
NBLKR = ROUND // K

_MESH = plsc.VectorSubcoreMesh(core_axis_name="c", subcore_axis_name="s",
                               num_cores=NC, num_subcores=NS)
_SC_PARAMS = pltpu.CompilerParams(needs_layout_passes=False)
_Z16 = functools.partial(jnp.zeros, (16,), jnp.float32)


def _tc_pre(x_ref, w_ref, awl_ref, awr_ref, ab_ref, h_ref, hl_ref, hr_ref):
    h = jnp.dot(x_ref[...], w_ref[...], preferred_element_type=jnp.float32)
    h_ref[...] = h
    hl_ref[...] = jnp.sum(h * awl_ref[...], axis=1, keepdims=True) + ab_ref[0]
    hr_ref[...] = jnp.sum(h * awr_ref[...], axis=1, keepdims=True)


def _sc_att(src_hbm, dst_hbm, hl_hbm, hr_hbm, e_hbm, hsum_hbm,
            src_v, dst_v, hl_v, hr_v, e_v, hsum_v, tmp_v, acc_v, hstage_sh):
    cid = lax.axis_index("c")
    sid = lax.axis_index("s")
    wid = cid * NS + sid
    base = wid * EPW
    pltpu.sync_copy(src_hbm.at[pl.ds(base, EPW)], src_v)
    pltpu.sync_copy(dst_hbm.at[pl.ds(base, EPW)], dst_v)
    pltpu.sync_copy(hl_hbm, hl_v)
    pltpu.sync_copy(hr_hbm, hr_v)

    def zero_hsum(k, carry):
        hsum_v[pl.ds(k * 16, 16)] = _Z16()
        return carry
    lax.fori_loop(0, NPAD // 16, zero_hsum, 0)

    def step(i, carry):
        sl = pl.ds(i * 16, 16)
        s16 = src_v[sl]
        d16 = dst_v[sl]
        t = plsc.load_gather(hl_v, [s16]) + plsc.load_gather(hr_v, [d16])
        t = jnp.where(t > 0.0, t, t * NEG_SLOPE)
        e16 = jnp.exp(t)
        e_v[sl] = e16
        plsc.addupdate_scatter(hsum_v, [s16], e16)
        return carry
    lax.fori_loop(0, EPW // 16, step, 0)

    pltpu.sync_copy(e_v, e_hbm.at[pl.ds(base, EPW)])
    pltpu.sync_copy(hsum_v, hstage_sh.at[sid])
    plsc.subcore_barrier()

    def zero_acc(k, carry):
        acc_v[pl.ds(k * 16, 16)] = _Z16()
        return carry
    lax.fori_loop(0, CHK // 16, zero_acc, 0)
    for r in range(NS):
        pltpu.sync_copy(hstage_sh.at[r, pl.ds(sid * CHK, CHK)], tmp_v)

        def addc(k, carry):
            sl = pl.ds(k * 16, 16)
            acc_v[sl] = acc_v[sl] + tmp_v[sl]
            return carry
        lax.fori_loop(0, CHK // 16, addc, 0)
    pltpu.sync_copy(acc_v, hsum_hbm.at[cid, pl.ds(sid * CHK, CHK)])


def _sc_agg(h_hbm, src_hbm, dstr_hbm, e_hbm, adj_hbm, hsum_hbm,
            alpha_hbm, outp_hbm,
            src_v, dst2_v, e_v, adj_v, hs_v, tmp_v, alpha_v,
            rowbuf_v, out_sh, gsem, ssem):
    cid = lax.axis_index("c")
    sid = lax.axis_index("s")
    wid = cid * NS + sid
    base = wid * EPW

    pltpu.sync_copy(hsum_hbm.at[0], hs_v)
    for t in range(NPAD // CHK):
        pltpu.sync_copy(hsum_hbm.at[1, pl.ds(t * CHK, CHK)], tmp_v)

        def addp(k, carry, t=t):
            sl = pl.ds(t * CHK + k * 16, 16)
            sl2 = pl.ds(k * 16, 16)
            hs_v[sl] = hs_v[sl] + tmp_v[sl2]
            return carry
        lax.fori_loop(0, CHK // 16, addp, 0)

    def zrow(r, carry):
        for c in range(NH // 16):
            rowbuf_v[0, r, pl.ds(c * 16, 16)] = _Z16()
        return carry
    lax.fori_loop(0, K, zrow, 0)
    for t in range(CHK // K):
        pltpu.sync_copy(rowbuf_v.at[0],
                        out_sh.at[pl.ds(sid * CHK + t * K, K)])
    plsc.subcore_barrier()

    def rnd(r, carry):
        rbase = base + r * ROUND
        pltpu.sync_copy(src_hbm.at[pl.ds(rbase, ROUND)], src_v)
        pltpu.sync_copy(dstr_hbm.at[wid, r], dst2_v)
        pltpu.sync_copy(e_hbm.at[pl.ds(rbase, ROUND)], e_v)
        pltpu.sync_copy(adj_hbm.at[pl.ds(rbase, ROUND)], adj_v)

        pltpu.async_copy(h_hbm.at[dst2_v.at[0]], rowbuf_v.at[0], gsem.at[0])

        def blk(j, carry):
            p = lax.rem(j, 2)
            pltpu.make_async_copy(h_hbm.at[dst2_v.at[j]], rowbuf_v.at[p],
                                  gsem.at[p]).wait()

            @pl.when(j + 1 < NBLKR)
            def _prefetch():
                @pl.when(j > 0)
                def _drain():
                    for _ in range(SB):
                        pltpu.make_async_copy(
                            rowbuf_v.at[1 - p, pl.ds(0, 16)],
                            out_sh.at[pl.ds(0, 16)], ssem).wait()
                pltpu.async_copy(h_hbm.at[dst2_v.at[j + 1]],
                                 rowbuf_v.at[1 - p], gsem.at[1 - p])

            for sb in range(SB):
                sl = pl.ds(j * K + sb * 16, 16)
                s16 = src_v[sl]
                al16 = e_v[sl] / plsc.load_gather(hs_v, [s16])
                alpha_v[sl] = al16
                val16 = al16 * adj_v[sl]
                for rr16 in range(16):
                    sp = jnp.broadcast_to(val16[rr16], (16,))
                    rr = sb * 16 + rr16
                    for c in range(NH // 16):
                        csl = pl.ds(c * 16, 16)
                        rowbuf_v[p, rr, csl] = rowbuf_v[p, rr, csl] * sp
                pltpu.async_copy(rowbuf_v.at[p, pl.ds(sb * 16, 16)],
                                 out_sh.at[s16], ssem, add=True)
            return carry
        lax.fori_loop(0, NBLKR, blk, 0)
        for _ in range(2 * SB):
            pltpu.make_async_copy(rowbuf_v.at[0, pl.ds(0, 16)],
                                  out_sh.at[pl.ds(0, 16)], ssem).wait()
        pltpu.sync_copy(alpha_v, alpha_hbm.at[pl.ds(rbase, ROUND)])
        return carry
    lax.fori_loop(0, NR, rnd, 0)

    plsc.subcore_barrier()
    pltpu.sync_copy(out_sh.at[pl.ds(sid * CHK, CHK)],
                    outp_hbm.at[cid, pl.ds(sid * CHK, CHK)])


def _tc_fin(p_ref, o_ref):
    o_ref[...] = p_ref[0, :N, :] + p_ref[1, :N, :]


@jax.jit
def kernel(x, edge_index, adj_vals, W_fc, a_W, a_b):
    src = edge_index[0]
    dst = edge_index[1]
    awl = a_W[:NH, 0].reshape(1, NH)
    awr = a_W[NH:, 0].reshape(1, NH)

    h, hl2, hr2 = pl.pallas_call(
        _tc_pre,
        out_shape=[
            jax.ShapeDtypeStruct((N, NH), jnp.float32),
            jax.ShapeDtypeStruct((N, 1), jnp.float32),
            jax.ShapeDtypeStruct((N, 1), jnp.float32),
        ],
        in_specs=[
            pl.BlockSpec(memory_space=pltpu.MemorySpace.VMEM),
            pl.BlockSpec(memory_space=pltpu.MemorySpace.VMEM),
            pl.BlockSpec(memory_space=pltpu.MemorySpace.VMEM),
            pl.BlockSpec(memory_space=pltpu.MemorySpace.VMEM),
            pl.BlockSpec(memory_space=pltpu.MemorySpace.SMEM),
        ],
    )(x, W_fc, awl, awr, a_b)
    hl = hl2.reshape(N)
    hr = hr2.reshape(N)

    e, hsum = pl.kernel(
        _sc_att,
        out_type=[
            jax.ShapeDtypeStruct((E,), jnp.float32),
            jax.ShapeDtypeStruct((NC, NPAD), jnp.float32),
        ],
        mesh=_MESH,
        compiler_params=_SC_PARAMS,
        scratch_types=[
            pltpu.VMEM((EPW,), jnp.int32),
            pltpu.VMEM((EPW,), jnp.int32),
            pltpu.VMEM((N,), jnp.float32),
            pltpu.VMEM((N,), jnp.float32),
            pltpu.VMEM((EPW,), jnp.float32),
            pltpu.VMEM((NPAD,), jnp.float32),
            pltpu.VMEM((CHK,), jnp.float32),
            pltpu.VMEM((CHK,), jnp.float32),
            pltpu.MemorySpace.VMEM_SHARED((NS, NPAD), jnp.float32),
        ],
    )(src, dst, hl, hr)

    dst_r = dst.reshape(NW, NR, NBLKR, K)
    alpha, outp = pl.kernel(
        _sc_agg,
        out_type=[
            jax.ShapeDtypeStruct((E,), jnp.float32),
            jax.ShapeDtypeStruct((NC, NPAD, NH), jnp.float32),
        ],
        mesh=_MESH,
        compiler_params=_SC_PARAMS,
        scratch_types=[
            pltpu.VMEM((ROUND,), jnp.int32),
            pltpu.VMEM((NBLKR, K), jnp.int32),
            pltpu.VMEM((ROUND,), jnp.float32),
            pltpu.VMEM((ROUND,), jnp.float32),
            pltpu.VMEM((NPAD,), jnp.float32),
            pltpu.VMEM((CHK,), jnp.float32),
            pltpu.VMEM((ROUND,), jnp.float32),
            pltpu.VMEM((2, K, NH), jnp.float32),
            pltpu.MemorySpace.VMEM_SHARED((NPAD, NH), jnp.float32),
            pltpu.SemaphoreType.DMA((2,)),
            pltpu.SemaphoreType.DMA,
        ],
    )(h, src, dst_r, e, adj_vals, hsum)

    out = pl.pallas_call(
        _tc_fin,
        out_shape=jax.ShapeDtypeStruct((N, NH), jnp.float32),
    )(outp)
    return (out, alpha)


# speedup vs baseline: 19.1642x; 1.0068x over previous
import functools

import jax
import jax.numpy as jnp
from jax import lax
from jax.experimental import pallas as pl
from jax.experimental.pallas import tpu as pltpu
from jax.experimental.pallas import tpu_sc as plsc

N = 10000
E = 320000
NF = 128
NH = 128
NEG_SLOPE = 0.05

NC = 2
NS = 16
NW = NC * NS
EPW = E // NW
NPAD = 10240
CHK = NPAD // NS
K = 80
SB = K // 16
NR = 5
ROUND = EPW // NR
NBLKR = ROUND // K
NBUF = 3
NLAST = N - (NS - 1) * CHK

_MESH = plsc.VectorSubcoreMesh(core_axis_name="c", subcore_axis_name="s",
                               num_cores=NC, num_subcores=NS)
_SC_PARAMS = pltpu.CompilerParams(needs_layout_passes=False)
_Z16 = functools.partial(jnp.zeros, (16,), jnp.float32)


def _tc_pre(x_ref, w_ref, awl_ref, awr_ref, ab_ref, h_ref, hl_ref, hr_ref):
    h = jnp.dot(x_ref[...], w_ref[...], preferred_element_type=jnp.float32)
    h_ref[...] = h
    hl_ref[...] = jnp.sum(h * awl_ref[...], axis=1, keepdims=True) + ab_ref[0]
    hr_ref[...] = jnp.sum(h * awr_ref[...], axis=1, keepdims=True)


def _sc_att(src_hbm, dst_hbm, hl_hbm, hr_hbm, e_hbm, hsum_hbm,
            src_v, dst_v, hl_v, hr_v, e_v, hsum_v, tmp_v, acc_v, hstage_sh):
    cid = lax.axis_index("c")
    sid = lax.axis_index("s")
    wid = cid * NS + sid
    base = wid * EPW
    pltpu.sync_copy(src_hbm.at[pl.ds(base, EPW)], src_v)
    pltpu.sync_copy(dst_hbm.at[pl.ds(base, EPW)], dst_v)
    pltpu.sync_copy(hl_hbm, hl_v)
    pltpu.sync_copy(hr_hbm, hr_v)

    def zero_hsum(k, carry):
        hsum_v[pl.ds(k * 16, 16)] = _Z16()
        return carry
    lax.fori_loop(0, NPAD // 16, zero_hsum, 0)

    def step(i, carry):
        sl = pl.ds(i * 16, 16)
        s16 = src_v[sl]
        d16 = dst_v[sl]
        t = plsc.load_gather(hl_v, [s16]) + plsc.load_gather(hr_v, [d16])
        t = jnp.where(t > 0.0, t, t * NEG_SLOPE)
        e16 = jnp.exp(t)
        e_v[sl] = e16
        plsc.addupdate_scatter(hsum_v, [s16], e16)
        return carry
    lax.fori_loop(0, EPW // 16, step, 0)

    pltpu.sync_copy(e_v, e_hbm.at[pl.ds(base, EPW)])
    pltpu.sync_copy(hsum_v, hstage_sh.at[sid])
    plsc.subcore_barrier()

    def zero_acc(k, carry):
        acc_v[pl.ds(k * 16, 16)] = _Z16()
        return carry
    lax.fori_loop(0, CHK // 16, zero_acc, 0)
    for r in range(NS):
        pltpu.sync_copy(hstage_sh.at[r, pl.ds(sid * CHK, CHK)], tmp_v)

        def addc(k, carry):
            sl = pl.ds(k * 16, 16)
            acc_v[sl] = acc_v[sl] + tmp_v[sl]
            return carry
        lax.fori_loop(0, CHK // 16, addc, 0)
    pltpu.sync_copy(acc_v, hsum_hbm.at[pl.ds(cid * NPAD + sid * CHK, CHK)])


def _sc_agg(h_hbm, src_hbm, dstr_hbm, e_hbm, adj_hbm, hsum_hbm,
            alpha_hbm, outp_hbm,
            src_v, dst2_v, e_v, adj_v, hs_v,
            rowbuf_v, out_sh, gsem, ssem):
    cid = lax.axis_index("c")
    sid = lax.axis_index("s")
    wid = cid * NS + sid
    base = wid * EPW

    pltpu.sync_copy(hsum_hbm.at[pl.ds(0, N)], hs_v)
    for t in range(NS):
        cnt = CHK if (t + 1) * CHK <= N else N - t * CHK
        pltpu.sync_copy(hsum_hbm.at[pl.ds(NPAD + t * CHK, cnt)],
                        adj_v.at[pl.ds(0, cnt)])

        def addp(k, carry, t=t):
            sl = pl.ds(t * CHK + k * 16, 16)
            sl2 = pl.ds(k * 16, 16)
            hs_v[sl] = hs_v[sl] + adj_v[sl2]
            return carry
        lax.fori_loop(0, cnt // 16, addp, 0)

    def zrow(r, carry):
        for c in range(NH // 16):
            rowbuf_v[0, r, pl.ds(c * 16, 16)] = _Z16()
        return carry
    lax.fori_loop(0, K, zrow, 0)
    for t in range(CHK // K):
        if t * K < NLAST:
            pltpu.sync_copy(rowbuf_v.at[0],
                            out_sh.at[pl.ds(sid * CHK + t * K, K)])
        else:
            @pl.when(sid < NS - 1)
            def _ztail(t=t):
                pltpu.sync_copy(rowbuf_v.at[0],
                                out_sh.at[pl.ds(sid * CHK + t * K, K)])
    plsc.subcore_barrier()

    def rnd(r, carry):
        rbase = base + r * ROUND
        pltpu.sync_copy(src_hbm.at[pl.ds(rbase, ROUND)], src_v)
        pltpu.sync_copy(dstr_hbm.at[wid, r], dst2_v)
        pltpu.sync_copy(e_hbm.at[pl.ds(rbase, ROUND)], e_v)
        pltpu.sync_copy(adj_hbm.at[pl.ds(rbase, ROUND)], adj_v)

        pltpu.async_copy(h_hbm.at[dst2_v.at[0]], rowbuf_v.at[0], gsem.at[0])

        def blk(j, carry):
            p = lax.rem(j, NBUF)
            pltpu.make_async_copy(h_hbm.at[dst2_v.at[j]], rowbuf_v.at[p],
                                  gsem.at[p]).wait()

            @pl.when(j + 1 < NBLKR)
            def _prefetch():
                @pl.when(j >= NBUF - 1)
                def _drain():
                    for _ in range(SB):
                        pltpu.make_async_copy(
                            rowbuf_v.at[0, pl.ds(0, 16)],
                            out_sh.at[pl.ds(0, 16)], ssem).wait()
                q = lax.rem(j + 1, NBUF)
                pltpu.async_copy(h_hbm.at[dst2_v.at[j + 1]],
                                 rowbuf_v.at[q], gsem.at[q])

            for sb in range(SB):
                sl = pl.ds(j * K + sb * 16, 16)
                s16 = src_v[sl]
                al16 = e_v[sl] / plsc.load_gather(hs_v, [s16])
                e_v[sl] = al16
                val16 = al16 * adj_v[sl]
                for rr16 in range(16):
                    sp = jnp.broadcast_to(val16[rr16], (16,))
                    rr = sb * 16 + rr16
                    for c in range(NH // 16):
                        csl = pl.ds(c * 16, 16)
                        rowbuf_v[p, rr, csl] = rowbuf_v[p, rr, csl] * sp
                pltpu.async_copy(rowbuf_v.at[p, pl.ds(sb * 16, 16)],
                                 out_sh.at[s16], ssem, add=True)
            return carry
        lax.fori_loop(0, NBLKR, blk, 0)
        for _ in range(NBUF * SB):
            pltpu.make_async_copy(rowbuf_v.at[0, pl.ds(0, 16)],
                                  out_sh.at[pl.ds(0, 16)], ssem).wait()
        pltpu.sync_copy(e_v, alpha_hbm.at[pl.ds(rbase, ROUND)])
        return carry
    lax.fori_loop(0, NR, rnd, 0)

    plsc.subcore_barrier()

    @pl.when(sid < NS - 1)
    def _wr():
        pltpu.sync_copy(out_sh.at[pl.ds(sid * CHK, CHK)],
                        outp_hbm.at[cid, pl.ds(sid * CHK, CHK)])

    @pl.when(sid == NS - 1)
    def _wr_tail():
        pltpu.sync_copy(out_sh.at[pl.ds(sid * CHK, NLAST)],
                        outp_hbm.at[cid, pl.ds(sid * CHK, NLAST)])


def _tc_fin(p_ref, o_ref):
    o_ref[...] = p_ref[0, :N, :] + p_ref[1, :N, :]


@jax.jit
def kernel(x, edge_index, adj_vals, W_fc, a_W, a_b):
    src = edge_index[0]
    dst = edge_index[1]
    awl = a_W[:NH, 0].reshape(1, NH)
    awr = a_W[NH:, 0].reshape(1, NH)

    h, hl2, hr2 = pl.pallas_call(
        _tc_pre,
        out_shape=[
            jax.ShapeDtypeStruct((N, NH), jnp.float32),
            jax.ShapeDtypeStruct((N, 1), jnp.float32),
            jax.ShapeDtypeStruct((N, 1), jnp.float32),
        ],
        in_specs=[
            pl.BlockSpec(memory_space=pltpu.MemorySpace.VMEM),
            pl.BlockSpec(memory_space=pltpu.MemorySpace.VMEM),
            pl.BlockSpec(memory_space=pltpu.MemorySpace.VMEM),
            pl.BlockSpec(memory_space=pltpu.MemorySpace.VMEM),
            pl.BlockSpec(memory_space=pltpu.MemorySpace.SMEM),
        ],
    )(x, W_fc, awl, awr, a_b)
    hl = hl2.reshape(N)
    hr = hr2.reshape(N)

    e, hsum = pl.kernel(
        _sc_att,
        out_type=[
            jax.ShapeDtypeStruct((E,), jnp.float32),
            jax.ShapeDtypeStruct((NC * NPAD,), jnp.float32),
        ],
        mesh=_MESH,
        compiler_params=_SC_PARAMS,
        scratch_types=[
            pltpu.VMEM((EPW,), jnp.int32),
            pltpu.VMEM((EPW,), jnp.int32),
            pltpu.VMEM((N,), jnp.float32),
            pltpu.VMEM((N,), jnp.float32),
            pltpu.VMEM((EPW,), jnp.float32),
            pltpu.VMEM((NPAD,), jnp.float32),
            pltpu.VMEM((CHK,), jnp.float32),
            pltpu.VMEM((CHK,), jnp.float32),
            pltpu.MemorySpace.VMEM_SHARED((NS, NPAD), jnp.float32),
        ],
    )(src, dst, hl, hr)

    dst_r = dst.reshape(NW, NR, NBLKR, K)
    alpha, outp = pl.kernel(
        _sc_agg,
        out_type=[
            jax.ShapeDtypeStruct((E,), jnp.float32),
            jax.ShapeDtypeStruct((NC, N, NH), jnp.float32),
        ],
        mesh=_MESH,
        compiler_params=_SC_PARAMS,
        scratch_types=[
            pltpu.VMEM((ROUND,), jnp.int32),
            pltpu.VMEM((NBLKR, K), jnp.int32),
            pltpu.VMEM((ROUND,), jnp.float32),
            pltpu.VMEM((ROUND,), jnp.float32),
            pltpu.VMEM((N,), jnp.float32),
            pltpu.VMEM((NBUF, K, NH), jnp.float32),
            pltpu.MemorySpace.VMEM_SHARED((N, NH), jnp.float32),
            pltpu.SemaphoreType.DMA((NBUF,)),
            pltpu.SemaphoreType.DMA,
        ],
    )(h, src, dst_r, e, adj_vals, hsum)

    out = pl.pallas_call(
        _tc_fin,
        out_shape=jax.ShapeDtypeStruct((N, NH), jnp.float32),
    )(outp)
    return (out, alpha)
